# baseline (speedup 1.0000x reference)
import functools

import jax
import jax.numpy as jnp
from jax import lax
from jax.experimental import pallas as pl
from jax.experimental.pallas import tpu as pltpu
from jax.experimental.pallas import tpu_sc as plsc

_NUM_MEMORY = 100000
_NUM_DIMS = 64
_BATCH = 1024
_HIST = 50

_NC = 2
_NS = 16
_L = 16
_NW = _NC * _NS
_B_PER_W = _BATCH // _NW
_NIDX = _BATCH * _HIST
_CHUNK = 3136
_MASK_PAD = _NW * _CHUNK


@functools.partial(
    pl.kernel,
    mesh=plsc.VectorSubcoreMesh(core_axis_name="c", subcore_axis_name="s"),
    compiler_params=pltpu.CompilerParams(
        needs_layout_passes=False, use_tc_tiling_on_sc=False
    ),
    out_type=[
        jax.ShapeDtypeStruct((_BATCH, _NUM_DIMS), jnp.float32),
        jax.ShapeDtypeStruct((_MASK_PAD,), jnp.int32),
    ],
    scratch_types=[
        pltpu.VMEM((_B_PER_W,), jnp.int32),
        pltpu.VMEM((_B_PER_W, _NUM_DIMS), jnp.float32),
        pltpu.VMEM((_BATCH, _HIST), jnp.int32),
        pltpu.VMEM((_CHUNK,), jnp.int32),
        pltpu.SemaphoreType.DMA,
    ],
)
def _sc_kernel(src_hbm, tgt_hbm, mem_hbm, vecs_hbm, mask_hbm,
               sidx_v, rows_v, tidx_v, chunk_v, sem):
    wid = lax.axis_index("s") * _NC + lax.axis_index("c")

    base = wid * _B_PER_W
    pltpu.sync_copy(src_hbm.at[pl.ds(base, _B_PER_W)], sidx_v)
    pltpu.async_copy(mem_hbm.at[sidx_v], rows_v, sem).wait()
    pltpu.sync_copy(rows_v, vecs_hbm.at[pl.ds(base, _B_PER_W)])

    pltpu.sync_copy(tgt_hbm, tidx_v)

    zeros = jnp.zeros((_L,), jnp.int32)
    ones = jnp.ones((_L,), jnp.int32)

    def _zero_body(i, carry):
        chunk_v[pl.ds(i * _L, _L)] = zeros
        return carry

    lax.fori_loop(0, _CHUNK // _L, _zero_body, 0)

    lo = wid * _CHUNK
    lo_v = jnp.full((_L,), 0, jnp.int32) + lo

    k_iota = lax.iota(jnp.int32, _L)

    def _scat_body(i, carry):
        k = k_iota + i * _L
        r = k // _HIST
        c = k - r * _HIST
        v = plsc.load_gather(tidx_v, [r, c])
        local = v - lo_v
        m = (local >= 0) & (local < _CHUNK)
        safe = jnp.where(m, local, 0)
        plsc.store_scatter(chunk_v, [safe], ones, mask=m)
        return carry

    lax.fori_loop(0, _NIDX // _L, _scat_body, 0)

    pltpu.sync_copy(chunk_v, mask_hbm.at[pl.ds(lo, _CHUNK)])


def kernel(src_ids, tgt_ids, memory):
    vecs, mask_i32 = _sc_kernel(src_ids, tgt_ids, memory)
    connected_mask = mask_i32[:_NUM_MEMORY].astype(jnp.bool_)
    return (vecs, connected_mask)


# speedup vs baseline: 1.5884x; 1.5884x over previous
import functools

import jax
import jax.numpy as jnp
from jax import lax
from jax.experimental import pallas as pl
from jax.experimental.pallas import tpu as pltpu
from jax.experimental.pallas import tpu_sc as plsc

_NUM_MEMORY = 100000
_NUM_DIMS = 64
_BATCH = 1024
_HIST = 50
_HIST_PAD = 64

_NC = 2
_NS = 16
_L = 16
_NW = _NC * _NS
_B_PER_W = _BATCH // _NW
_SLICE = 6272
_MASK_PAD = _NS * _SLICE


@functools.partial(
    pl.kernel,
    mesh=plsc.VectorSubcoreMesh(core_axis_name="c", subcore_axis_name="s"),
    out_type=jax.ShapeDtypeStruct((_BATCH, _NUM_DIMS), jnp.float32),
    scratch_types=[
        pltpu.VMEM((_B_PER_W,), jnp.int32),
        pltpu.SemaphoreType.DMA,
    ],
)
def _sc_gather(src_hbm, mem_hbm, vecs_hbm, sidx_v, sem):
    wid = lax.axis_index("s") * _NC + lax.axis_index("c")
    base = wid * _B_PER_W
    pltpu.sync_copy(src_hbm.at[pl.ds(base, _B_PER_W)], sidx_v)
    copies = []
    for g in range(_B_PER_W // _L):
        v = sidx_v[pl.ds(g * _L, _L)]
        for j in range(_L):
            b = base + g * _L + j
            copies.append(
                pltpu.async_copy(mem_hbm.at[v[j]], vecs_hbm.at[b], sem)
            )
    for c in copies:
        c.wait()


@functools.partial(
    pl.kernel,
    mesh=plsc.VectorSubcoreMesh(core_axis_name="c", subcore_axis_name="s"),
    compiler_params=pltpu.CompilerParams(
        needs_layout_passes=False, use_tc_tiling_on_sc=False
    ),
    out_type=[
        jax.ShapeDtypeStruct((_MASK_PAD,), jnp.int32),
        jax.ShapeDtypeStruct((_MASK_PAD,), jnp.int32),
    ],
    scratch_types=[
        pltpu.VMEM((_B_PER_W, _HIST_PAD), jnp.int32),
        pltpu.VMEM((_HIST_PAD,), jnp.int32),
        pltpu.VMEM((_SLICE,), jnp.int32),
        pltpu.VMEM_SHARED((_MASK_PAD,), jnp.int32),
        pltpu.SemaphoreType.DMA,
    ],
)
def _sc_mask(tgt_hbm, m0_hbm, m1_hbm, stage_v, ones_v, zbuf_v, shared, sem):
    cid = lax.axis_index("c")
    sid = lax.axis_index("s")
    wid = sid * _NC + cid

    ones = jnp.ones((_L,), jnp.int32)
    zeros = jnp.zeros((_L,), jnp.int32)
    for j in range(_HIST_PAD // _L):
        ones_v[pl.ds(j * _L, _L)] = ones

    def _zero_body(i, carry):
        zbuf_v[pl.ds(i * _L, _L)] = zeros
        return carry

    lax.fori_loop(0, _SLICE // _L, _zero_body, 0)

    pltpu.sync_copy(tgt_hbm.at[pl.ds(wid * _B_PER_W, _B_PER_W), :], stage_v)

    lo = sid * _SLICE
    pltpu.sync_copy(zbuf_v, shared.at[pl.ds(lo, _SLICE)])
    plsc.subcore_barrier()

    copies = []
    for r in range(_B_PER_W):
        copies.append(
            pltpu.async_copy(ones_v, shared.at[stage_v.at[r]], sem)
        )
    for c in copies:
        c.wait()
    plsc.subcore_barrier()

    @pl.when(cid == 0)
    def _():
        pltpu.sync_copy(shared.at[pl.ds(lo, _SLICE)], m0_hbm.at[pl.ds(lo, _SLICE)])

    @pl.when(cid == 1)
    def _():
        pltpu.sync_copy(shared.at[pl.ds(lo, _SLICE)], m1_hbm.at[pl.ds(lo, _SLICE)])


def kernel(src_ids, tgt_ids, memory):
    vecs = _sc_gather(src_ids, memory)
    pad = (
        jnp.arange(_HIST_PAD - _HIST, dtype=jnp.int32)[None, :]
        + 16 * jnp.arange(_BATCH, dtype=jnp.int32)[:, None]
    ) % (_MASK_PAD - _NUM_MEMORY) + _NUM_MEMORY
    tgt_padded = jnp.concatenate([tgt_ids, pad], axis=1)
    m0, m1 = _sc_mask(tgt_padded)
    connected_mask = (m0 | m1)[:_NUM_MEMORY].astype(jnp.bool_)
    return (vecs, connected_mask)
